# Initial kernel scaffold; baseline (speedup 1.0000x reference)
import jax
import jax.numpy as jnp
from jax.experimental import pallas as pl


def kernel(tgt_chars, tgt_compos, ctx_words, noise_idx, word_emb, char_emb, compo_emb):
    raise NotImplementedError("write your pallas kernel here")


# speedup vs baseline: 1.0264x; 1.0264x over previous
import functools

import jax
import jax.numpy as jnp
from jax import lax
from jax.experimental import pallas as pl
from jax.experimental.pallas import tpu as pltpu
from jax.experimental.pallas import tpu_sc as plsc

B = 1024
DIM = 300
WIN = 20
K = 120
KPAD = 128
L = 16
NFULL = DIM // L
TAIL = DIM - NFULL * L


def _sc_geometry():
    try:
        info = plsc.get_sparse_core_info()
        return info.num_cores, info.num_subcores
    except Exception:
        return 2, 16


def _sc_logits(tgt_chars, tgt_compos8, all_idx, word_emb, char_emb, compo_emb):
    nc, ns = _sc_geometry()
    nw = nc * ns
    bpw = B // nw
    mesh = plsc.VectorSubcoreMesh(core_axis_name="c", subcore_axis_name="s",
                                  num_cores=nc, num_subcores=ns)

    @functools.partial(
        pl.kernel,
        out_type=jax.ShapeDtypeStruct((B, KPAD), jnp.float32),
        mesh=mesh,
        compiler_params=pltpu.CompilerParams(use_tc_tiling_on_sc=False,
                                             needs_layout_passes=False),
        scratch_types=[
            pltpu.VMEM((bpw, 8), jnp.int32),
            pltpu.VMEM((bpw, 8), jnp.int32),
            pltpu.VMEM((bpw, K), jnp.int32),
            pltpu.VMEM((8, DIM), jnp.float32),
            pltpu.VMEM((8, DIM), jnp.float32),
            pltpu.VMEM((K, DIM), jnp.float32),
            pltpu.VMEM((NFULL * L + L,), jnp.float32),
            pltpu.VMEM((bpw, KPAD), jnp.float32),
        ],
    )
    def k(chars_hbm, compos_hbm, aidx_hbm, word_hbm, char_hbm, compo_hbm,
          out_hbm, cidx_v, oidx_v, widx_v, crows, orows, wrows, tgt_v, log_v):
        wid = lax.axis_index("s") * nc + lax.axis_index("c")
        base = wid * bpw
        pltpu.sync_copy(chars_hbm.at[pl.ds(base, bpw)], cidx_v)
        pltpu.sync_copy(compos_hbm.at[pl.ds(base, bpw)], oidx_v)
        pltpu.sync_copy(aidx_hbm.at[pl.ds(base, bpw)], widx_v)

        lanes = lax.iota(jnp.int32, L)
        tailmask = jnp.where(lanes < TAIL, 1.0, 0.0).astype(jnp.float32)
        colt = jnp.minimum(lanes + NFULL * L, DIM - 1)

        def body(b, carry):
            pltpu.sync_copy(char_hbm.at[cidx_v.at[b]], crows)
            pltpu.sync_copy(compo_hbm.at[oidx_v.at[b]], orows)
            pltpu.sync_copy(word_hbm.at[widx_v.at[b]], wrows)

            for c in range(NFULL):
                s = crows[0, pl.ds(c * L, L)]
                for r in range(1, 8):
                    s = s + crows[r, pl.ds(c * L, L)]
                for r in range(4):
                    s = s + orows[r, pl.ds(c * L, L)]
                tgt_v[pl.ds(c * L, L)] = s
            s = plsc.load_gather(crows, [jnp.full((L,), 0, jnp.int32), colt])
            for r in range(1, 8):
                s = s + plsc.load_gather(
                    crows, [jnp.full((L,), r, jnp.int32), colt])
            for r in range(4):
                s = s + plsc.load_gather(
                    orows, [jnp.full((L,), r, jnp.int32), colt])
            tgt_v[pl.ds(NFULL * L, L)] = s * tailmask

            def dot_g(g, carry2):
                row_ids = jnp.minimum(g * L + lanes, K - 1)
                acc = jnp.zeros((L,), jnp.float32)
                for c in range(NFULL + 1):
                    tch = tgt_v[pl.ds(c * L, L)]
                    jmax = L if c < NFULL else TAIL
                    for j in range(jmax):
                        d = c * L + j
                        colv = jnp.full((L,), d, jnp.int32)
                        rv = plsc.load_gather(wrows, [row_ids, colv])
                        tb = jnp.broadcast_to(tch[j], (L,))
                        acc = acc + rv * tb
                log_v[b, pl.ds(g * L, L)] = acc
                return carry2

            lax.fori_loop(0, KPAD // L, dot_g, 0)
            return carry

        lax.fori_loop(0, bpw, body, 0)
        pltpu.sync_copy(log_v, out_hbm.at[pl.ds(base, bpw)])

    return k(tgt_chars, tgt_compos8, all_idx, word_emb, char_emb, compo_emb)


def _tc_loss(logits):
    def body(x_ref, o_ref):
        x = x_ref[...]
        col = lax.broadcasted_iota(jnp.int32, (B, KPAD), 1)
        sign = jnp.where(col < WIN, 1.0, -1.0).astype(jnp.float32)
        z = jax.nn.sigmoid(x * sign) + 1e-5
        v = jnp.where(col < K, jnp.log(z), 0.0)
        o_ref[...] = jnp.broadcast_to(-jnp.sum(v) / B, (1, 1))

    return pl.pallas_call(
        body, out_shape=jax.ShapeDtypeStruct((1, 1), jnp.float32))(logits)


def kernel(tgt_chars, tgt_compos, ctx_words, noise_idx,
           word_emb, char_emb, compo_emb):
    tgt_chars = tgt_chars.astype(jnp.int32)
    compos8 = jnp.zeros((B, 8), jnp.int32).at[:, :4].set(
        tgt_compos.astype(jnp.int32))
    all_idx = jnp.concatenate(
        [ctx_words.astype(jnp.int32), noise_idx.astype(jnp.int32)], axis=1)
    logits = _sc_logits(tgt_chars, compos8, all_idx,
                        word_emb, char_emb, compo_emb)
    return _tc_loss(logits)[0, 0]
